# Initial kernel scaffold; baseline (speedup 1.0000x reference)
import jax
import jax.numpy as jnp
from jax.experimental import pallas as pl


def kernel(level_indices, type_indices, feature_indices, exchange_indices, pair_indices, level_table, type_table, feature_table, exchange_table, pair_table, W, b):
    raise NotImplementedError("write your pallas kernel here")


# speedup vs baseline: 9.5145x; 9.5145x over previous
import functools

import jax
import jax.numpy as jnp
from jax import lax
from jax.experimental import pallas as pl
from jax.experimental.pallas import tpu as pltpu
from jax.experimental.pallas import tpu_sc as plsc

N_COLS = 65536
D = 128
ATTR = 25
REM = 28

NC = 2
NS = 16
NW = NC * NS
ROWS_PER_W = N_COLS // NW
CH = 128
NCHUNK = ROWS_PER_W // CH


def _proj_body(lvl_ref, typ_ref, feat_ref, exch_ref, pair_ref, w_ref, b_ref,
               li_ref, ti_ref, fi_ref, ei_ref,
               lt_ref, fe_ref, pp_ref, lti_ref, fei_ref):
    w = w_ref[...]
    dn = (((1,), (1,)), ((), ()))
    a_l = lax.dot_general(lvl_ref[...], w[:, 0:ATTR], dn,
                          preferred_element_type=jnp.float32)
    a_t = lax.dot_general(typ_ref[...], w[:, ATTR:2 * ATTR], dn,
                          preferred_element_type=jnp.float32)
    a_f = lax.dot_general(feat_ref[...], w[:, 2 * ATTR:3 * ATTR], dn,
                          preferred_element_type=jnp.float32)
    a_e = lax.dot_general(exch_ref[...], w[:, 3 * ATTR:4 * ATTR], dn,
                          preferred_element_type=jnp.float32)
    lt_ref[...] = a_l[:, None, :] + a_t[None, :, :] + b_ref[...][None, :, :]
    fe_ref[...] = a_f[:, None, :] + a_e[None, :, :]
    pp_ref[...] = lax.dot_general(pair_ref[...], w[:, 4 * ATTR:D], dn,
                                  preferred_element_type=jnp.float32)
    lti_ref[...] = li_ref[...] * 8 + ti_ref[...]
    fei_ref[...] = fi_ref[...] * 16 + ei_ref[...]


_proj_call = pl.pallas_call(
    _proj_body,
    out_shape=[
        jax.ShapeDtypeStruct((50, 8, D), jnp.float32),
        jax.ShapeDtypeStruct((32, 16, D), jnp.float32),
        jax.ShapeDtypeStruct((4096, D), jnp.float32),
        jax.ShapeDtypeStruct((N_COLS // D, D), jnp.int32),
        jax.ShapeDtypeStruct((N_COLS // D, D), jnp.int32),
    ],
)


def _sc_body(lti_hbm, fei_hbm, pi_hbm, plt_hbm, pfe_hbm, ppair_hbm, out_hbm,
             idx1, idx2, idx3, buf1, buf2, buf3, sem1, sem2, sem3):
    wid = lax.axis_index("s") * NC + lax.axis_index("c")
    base = wid * ROWS_PER_W

    def chunk_body(c, carry):
        off = pl.multiple_of(base + c * CH, CH)
        pltpu.sync_copy(lti_hbm.at[pl.ds(off, CH)], idx1)
        pltpu.sync_copy(fei_hbm.at[pl.ds(off, CH)], idx2)
        pltpu.sync_copy(pi_hbm.at[pl.ds(off, CH)], idx3)
        cp1 = pltpu.async_copy(plt_hbm.at[idx1], buf1, sem1)
        cp2 = pltpu.async_copy(pfe_hbm.at[idx2], buf2, sem2)
        cp3 = pltpu.async_copy(ppair_hbm.at[idx3], buf3, sem3)
        cp1.wait()
        cp2.wait()
        cp3.wait()

        def row_body(r, rcarry):
            for j in range(D // 16):
                sl = pl.ds(j * 16, 16)
                buf1[r, sl] = buf1[r, sl] + buf2[r, sl] + buf3[r, sl]
            return rcarry

        lax.fori_loop(0, CH, row_body, 0)
        pltpu.sync_copy(buf1, out_hbm.at[pl.ds(off, CH)])
        return carry

    lax.fori_loop(0, NCHUNK, chunk_body, 0)


_sc_call = functools.partial(
    pl.kernel,
    mesh=plsc.VectorSubcoreMesh(core_axis_name="c", subcore_axis_name="s"),
    out_type=jax.ShapeDtypeStruct((N_COLS, D), jnp.float32),
    scratch_types=[
        pltpu.VMEM((CH,), jnp.int32),
        pltpu.VMEM((CH,), jnp.int32),
        pltpu.VMEM((CH,), jnp.int32),
        pltpu.VMEM((CH, D), jnp.float32),
        pltpu.VMEM((CH, D), jnp.float32),
        pltpu.VMEM((CH, D), jnp.float32),
        pltpu.SemaphoreType.DMA,
        pltpu.SemaphoreType.DMA,
        pltpu.SemaphoreType.DMA,
    ],
)(_sc_body)


def kernel(level_indices, type_indices, feature_indices, exchange_indices,
           pair_indices, level_table, type_table, feature_table,
           exchange_table, pair_table, W, b):
    li = level_indices.astype(jnp.int32).reshape(N_COLS // D, D)
    ti = type_indices.astype(jnp.int32).reshape(N_COLS // D, D)
    fi = feature_indices.astype(jnp.int32).reshape(N_COLS // D, D)
    ei = exchange_indices.astype(jnp.int32).reshape(N_COLS // D, D)
    lt3, fe3, pp, lti, fei = _proj_call(
        level_table, type_table, feature_table, exchange_table, pair_table,
        W, b.reshape(1, D), li, ti, fi, ei)
    out = _sc_call(
        lti.reshape(N_COLS), fei.reshape(N_COLS),
        pair_indices.astype(jnp.int32),
        lt3.reshape(400, D), fe3.reshape(512, D), pp)
    return out


# speedup vs baseline: 13.6065x; 1.4301x over previous
import functools

import jax
import jax.numpy as jnp
from jax import lax
from jax.experimental import pallas as pl
from jax.experimental.pallas import tpu as pltpu
from jax.experimental.pallas import tpu_sc as plsc

N_COLS = 65536
D = 128
ATTR = 25
REM = 28

NC = 2
NS = 16
NW = NC * NS
ROWS_PER_W = N_COLS // NW
CH = 128
NCHUNK = ROWS_PER_W // CH


def _proj_body(lvl_ref, typ_ref, feat_ref, exch_ref, pair_ref, w_ref, b_ref,
               li_ref, ti_ref, fi_ref, ei_ref,
               lt_ref, fe_ref, pp_ref, lti_ref, fei_ref):
    w = w_ref[...]
    dn = (((1,), (1,)), ((), ()))
    a_l = lax.dot_general(lvl_ref[...], w[:, 0:ATTR], dn,
                          preferred_element_type=jnp.float32)
    a_t = lax.dot_general(typ_ref[...], w[:, ATTR:2 * ATTR], dn,
                          preferred_element_type=jnp.float32)
    a_f = lax.dot_general(feat_ref[...], w[:, 2 * ATTR:3 * ATTR], dn,
                          preferred_element_type=jnp.float32)
    a_e = lax.dot_general(exch_ref[...], w[:, 3 * ATTR:4 * ATTR], dn,
                          preferred_element_type=jnp.float32)
    lt_ref[...] = a_l[:, None, :] + a_t[None, :, :] + b_ref[...][None, :, :]
    fe_ref[...] = a_f[:, None, :] + a_e[None, :, :]
    pp_ref[...] = lax.dot_general(pair_ref[...], w[:, 4 * ATTR:D], dn,
                                  preferred_element_type=jnp.float32)
    lti_ref[...] = li_ref[...] * 8 + ti_ref[...]
    fei_ref[...] = fi_ref[...] * 16 + ei_ref[...]


_proj_call = pl.pallas_call(
    _proj_body,
    out_shape=[
        jax.ShapeDtypeStruct((50, 8, D), jnp.float32),
        jax.ShapeDtypeStruct((32, 16, D), jnp.float32),
        jax.ShapeDtypeStruct((4096, D), jnp.float32),
        jax.ShapeDtypeStruct((N_COLS // D, D), jnp.int32),
        jax.ShapeDtypeStruct((N_COLS // D, D), jnp.int32),
    ],
)


NPAIR = NCHUNK // 2


def _sc_body(lti_hbm, fei_hbm, pi_hbm, plt_hbm, pfe_hbm, ppair_hbm, out_hbm,
             ia1, ia2, ia3, g1a, g2a, g3a, g1b, g2b, g3b,
             s1a, s2a, s3a, s1b, s2b, s3b, osa, osb):
    wid = lax.axis_index("s") * NC + lax.axis_index("c")
    base = wid * ROWS_PER_W

    pltpu.sync_copy(lti_hbm.at[pl.ds(base, ROWS_PER_W)], ia1)
    pltpu.sync_copy(fei_hbm.at[pl.ds(base, ROWS_PER_W)], ia2)
    pltpu.sync_copy(pi_hbm.at[pl.ds(base, ROWS_PER_W)], ia3)

    def g_copies(c, bufs, sems):
        off = pl.multiple_of(c * CH, CH)
        srcs = (plt_hbm.at[ia1.at[pl.ds(off, CH)]],
                pfe_hbm.at[ia2.at[pl.ds(off, CH)]],
                ppair_hbm.at[ia3.at[pl.ds(off, CH)]])
        return [pltpu.make_async_copy(s, b, m)
                for s, b, m in zip(srcs, bufs, sems)]

    def issue(c, bufs, sems):
        for cp in g_copies(c, bufs, sems):
            cp.start()

    def wait_g(c, bufs, sems):
        for cp in g_copies(c, bufs, sems):
            cp.wait()

    def out_copy(c, g1, osem):
        off = pl.multiple_of(c * CH, CH)
        return pltpu.make_async_copy(
            g1, out_hbm.at[pl.ds(base + off, CH)], osem)

    def compute(g1, g2, g3):
        def row(r, carry):
            for j in range(D // 16):
                sl = pl.ds(j * 16, 16)
                plsc.addupdate(g1.at[r, sl], g2[r, sl] + g3[r, sl])
            return carry
        lax.fori_loop(0, CH, row, 0)

    bufs_a, sems_a = (g1a, g2a, g3a), (s1a, s2a, s3a)
    bufs_b, sems_b = (g1b, g2b, g3b), (s1b, s2b, s3b)

    issue(0, bufs_a, sems_a)

    def pair(i, carry):
        c0 = 2 * i
        wait_g(c0, bufs_a, sems_a)

        @pl.when(i > 0)
        def _():
            out_copy(c0 - 1, g1b, osb).wait()

        issue(c0 + 1, bufs_b, sems_b)
        compute(g1a, g2a, g3a)
        out_copy(c0, g1a, osa).start()
        wait_g(c0 + 1, bufs_b, sems_b)
        out_copy(c0, g1a, osa).wait()

        @pl.when(i < NPAIR - 1)
        def _():
            issue(c0 + 2, bufs_a, sems_a)

        compute(g1b, g2b, g3b)
        out_copy(c0 + 1, g1b, osb).start()
        return carry

    lax.fori_loop(0, NPAIR, pair, 0)
    out_copy(NCHUNK - 1, g1b, osb).wait()


_sc_call = functools.partial(
    pl.kernel,
    mesh=plsc.VectorSubcoreMesh(core_axis_name="c", subcore_axis_name="s"),
    out_type=jax.ShapeDtypeStruct((N_COLS, D), jnp.float32),
    scratch_types=[
        pltpu.VMEM((ROWS_PER_W,), jnp.int32),
        pltpu.VMEM((ROWS_PER_W,), jnp.int32),
        pltpu.VMEM((ROWS_PER_W,), jnp.int32),
        pltpu.VMEM((CH, D), jnp.float32),
        pltpu.VMEM((CH, D), jnp.float32),
        pltpu.VMEM((CH, D), jnp.float32),
        pltpu.VMEM((CH, D), jnp.float32),
        pltpu.VMEM((CH, D), jnp.float32),
        pltpu.VMEM((CH, D), jnp.float32),
        pltpu.SemaphoreType.DMA,
        pltpu.SemaphoreType.DMA,
        pltpu.SemaphoreType.DMA,
        pltpu.SemaphoreType.DMA,
        pltpu.SemaphoreType.DMA,
        pltpu.SemaphoreType.DMA,
        pltpu.SemaphoreType.DMA,
        pltpu.SemaphoreType.DMA,
    ],
)(_sc_body)


def kernel(level_indices, type_indices, feature_indices, exchange_indices,
           pair_indices, level_table, type_table, feature_table,
           exchange_table, pair_table, W, b):
    li = level_indices.astype(jnp.int32).reshape(N_COLS // D, D)
    ti = type_indices.astype(jnp.int32).reshape(N_COLS // D, D)
    fi = feature_indices.astype(jnp.int32).reshape(N_COLS // D, D)
    ei = exchange_indices.astype(jnp.int32).reshape(N_COLS // D, D)
    lt3, fe3, pp, lti, fei = _proj_call(
        level_table, type_table, feature_table, exchange_table, pair_table,
        W, b.reshape(1, D), li, ti, fi, ei)
    out = _sc_call(
        lti.reshape(N_COLS), fei.reshape(N_COLS),
        pair_indices.astype(jnp.int32),
        lt3.reshape(400, D), fe3.reshape(512, D), pp)
    return out


# speedup vs baseline: 13.6137x; 1.0005x over previous
import functools

import jax
import jax.numpy as jnp
from jax import lax
from jax.experimental import pallas as pl
from jax.experimental.pallas import tpu as pltpu
from jax.experimental.pallas import tpu_sc as plsc

N_COLS = 65536
D = 128
ATTR = 25
REM = 28

NC = 2
NS = 16
NW = NC * NS
ROWS_PER_W = N_COLS // NW
CH = 128
NCHUNK = ROWS_PER_W // CH


def _proj_body(lvl_ref, typ_ref, feat_ref, exch_ref, pair_ref, w_ref, b_ref,
               li_ref, ti_ref, fi_ref, ei_ref,
               lt_ref, fe_ref, pp_ref, lti_ref, fei_ref):
    w = w_ref[...]
    dn = (((1,), (1,)), ((), ()))
    a_l = lax.dot_general(lvl_ref[...], w[:, 0:ATTR], dn,
                          preferred_element_type=jnp.float32)
    a_t = lax.dot_general(typ_ref[...], w[:, ATTR:2 * ATTR], dn,
                          preferred_element_type=jnp.float32)
    a_f = lax.dot_general(feat_ref[...], w[:, 2 * ATTR:3 * ATTR], dn,
                          preferred_element_type=jnp.float32)
    a_e = lax.dot_general(exch_ref[...], w[:, 3 * ATTR:4 * ATTR], dn,
                          preferred_element_type=jnp.float32)
    lt_ref[...] = (a_l[:, None, :] + a_t[None, :, :]
                   + b_ref[...][None, None, :]).reshape(400, D)
    fe_ref[...] = (a_f[:, None, :] + a_e[None, :, :]).reshape(512, D)
    pp_ref[...] = lax.dot_general(pair_ref[...], w[:, 4 * ATTR:D], dn,
                                  preferred_element_type=jnp.float32)
    lti_ref[...] = li_ref[...] * 8 + ti_ref[...]
    fei_ref[...] = fi_ref[...] * 16 + ei_ref[...]


_proj_call = pl.pallas_call(
    _proj_body,
    out_shape=[
        jax.ShapeDtypeStruct((400, D), jnp.float32),
        jax.ShapeDtypeStruct((512, D), jnp.float32),
        jax.ShapeDtypeStruct((4096, D), jnp.float32),
        jax.ShapeDtypeStruct((N_COLS,), jnp.int32),
        jax.ShapeDtypeStruct((N_COLS,), jnp.int32),
    ],
)


NPAIR = NCHUNK // 2


def _sc_body(lti_hbm, fei_hbm, pi_hbm, plt_hbm, pfe_hbm, ppair_hbm, out_hbm,
             ia1, ia2, ia3, g1a, g2a, g3a, g1b, g2b, g3b,
             s1a, s2a, s3a, s1b, s2b, s3b, osa, osb):
    wid = lax.axis_index("s") * NC + lax.axis_index("c")
    base = wid * ROWS_PER_W

    pltpu.sync_copy(lti_hbm.at[pl.ds(base, ROWS_PER_W)], ia1)
    pltpu.sync_copy(fei_hbm.at[pl.ds(base, ROWS_PER_W)], ia2)
    pltpu.sync_copy(pi_hbm.at[pl.ds(base, ROWS_PER_W)], ia3)

    def g_copies(c, bufs, sems):
        off = pl.multiple_of(c * CH, CH)
        srcs = (plt_hbm.at[ia1.at[pl.ds(off, CH)]],
                pfe_hbm.at[ia2.at[pl.ds(off, CH)]],
                ppair_hbm.at[ia3.at[pl.ds(off, CH)]])
        return [pltpu.make_async_copy(s, b, m)
                for s, b, m in zip(srcs, bufs, sems)]

    def issue(c, bufs, sems):
        for cp in g_copies(c, bufs, sems):
            cp.start()

    def wait_g(c, bufs, sems):
        for cp in g_copies(c, bufs, sems):
            cp.wait()

    def out_copy(c, g1, osem):
        off = pl.multiple_of(c * CH, CH)
        return pltpu.make_async_copy(
            g1, out_hbm.at[pl.ds(base + off, CH)], osem)

    def compute(g1, g2, g3):
        def row(r, carry):
            for j in range(D // 16):
                sl = pl.ds(j * 16, 16)
                plsc.addupdate(g1.at[r, sl], g2[r, sl] + g3[r, sl])
            return carry
        lax.fori_loop(0, CH, row, 0)

    bufs_a, sems_a = (g1a, g2a, g3a), (s1a, s2a, s3a)
    bufs_b, sems_b = (g1b, g2b, g3b), (s1b, s2b, s3b)

    issue(0, bufs_a, sems_a)

    def pair(i, carry):
        c0 = 2 * i
        wait_g(c0, bufs_a, sems_a)

        @pl.when(i > 0)
        def _():
            out_copy(c0 - 1, g1b, osb).wait()

        issue(c0 + 1, bufs_b, sems_b)
        compute(g1a, g2a, g3a)
        out_copy(c0, g1a, osa).start()
        wait_g(c0 + 1, bufs_b, sems_b)
        out_copy(c0, g1a, osa).wait()

        @pl.when(i < NPAIR - 1)
        def _():
            issue(c0 + 2, bufs_a, sems_a)

        compute(g1b, g2b, g3b)
        out_copy(c0 + 1, g1b, osb).start()
        return carry

    lax.fori_loop(0, NPAIR, pair, 0)
    out_copy(NCHUNK - 1, g1b, osb).wait()


_sc_call = functools.partial(
    pl.kernel,
    mesh=plsc.VectorSubcoreMesh(core_axis_name="c", subcore_axis_name="s"),
    out_type=jax.ShapeDtypeStruct((N_COLS, D), jnp.float32),
    scratch_types=[
        pltpu.VMEM((ROWS_PER_W,), jnp.int32),
        pltpu.VMEM((ROWS_PER_W,), jnp.int32),
        pltpu.VMEM((ROWS_PER_W,), jnp.int32),
        pltpu.VMEM((CH, D), jnp.float32),
        pltpu.VMEM((CH, D), jnp.float32),
        pltpu.VMEM((CH, D), jnp.float32),
        pltpu.VMEM((CH, D), jnp.float32),
        pltpu.VMEM((CH, D), jnp.float32),
        pltpu.VMEM((CH, D), jnp.float32),
        pltpu.SemaphoreType.DMA,
        pltpu.SemaphoreType.DMA,
        pltpu.SemaphoreType.DMA,
        pltpu.SemaphoreType.DMA,
        pltpu.SemaphoreType.DMA,
        pltpu.SemaphoreType.DMA,
        pltpu.SemaphoreType.DMA,
        pltpu.SemaphoreType.DMA,
    ],
)(_sc_body)


def kernel(level_indices, type_indices, feature_indices, exchange_indices,
           pair_indices, level_table, type_table, feature_table,
           exchange_table, pair_table, W, b):
    lt, fe, pp, lti, fei = _proj_call(
        level_table, type_table, feature_table, exchange_table, pair_table,
        W, b.reshape(1, D),
        level_indices.astype(jnp.int32), type_indices.astype(jnp.int32),
        feature_indices.astype(jnp.int32), exchange_indices.astype(jnp.int32))
    out = _sc_call(lti, fei, pair_indices.astype(jnp.int32), lt, fe, pp)
    return out


# speedup vs baseline: 19.0148x; 1.3967x over previous
import functools

import jax
import jax.numpy as jnp
from jax import lax
from jax.experimental import pallas as pl
from jax.experimental.pallas import tpu as pltpu
from jax.experimental.pallas import tpu_sc as plsc

N_COLS = 65536
D = 128
ATTR = 25
REM = 28

NC = 2
NS = 16
NW = NC * NS
ROWS_PER_W = N_COLS // NW
CH = 128
NCHUNK = ROWS_PER_W // CH
NPAIR = NCHUNK // 2


def _proj_body(lvl_ref, typ_ref, feat_ref, exch_ref, pair_ref, w_ref, b_ref,
               li_ref, ti_ref, fi_ref, ei_ref,
               lt_ref, fe_ref, pp_ref, lti_ref, fei_ref):
    w = w_ref[...]
    dn = (((1,), (1,)), ((), ()))
    a_l = lax.dot_general(lvl_ref[...], w[:, 0:ATTR], dn,
                          preferred_element_type=jnp.float32)
    a_t = lax.dot_general(typ_ref[...], w[:, ATTR:2 * ATTR], dn,
                          preferred_element_type=jnp.float32)
    a_f = lax.dot_general(feat_ref[...], w[:, 2 * ATTR:3 * ATTR], dn,
                          preferred_element_type=jnp.float32)
    a_e = lax.dot_general(exch_ref[...], w[:, 3 * ATTR:4 * ATTR], dn,
                          preferred_element_type=jnp.float32)
    lt_ref[...] = (a_l[:, None, :] + a_t[None, :, :]
                   + b_ref[...][None, None, :]).reshape(400, D)
    fe_ref[...] = (a_f[:, None, :] + a_e[None, :, :]).reshape(512, D)
    pp_ref[...] = lax.dot_general(pair_ref[...], w[:, 4 * ATTR:D], dn,
                                  preferred_element_type=jnp.float32)
    lti_ref[...] = li_ref[...] * 8 + ti_ref[...]
    fei_ref[...] = fi_ref[...] * 16 + ei_ref[...]


_proj_call = pl.pallas_call(
    _proj_body,
    out_shape=[
        jax.ShapeDtypeStruct((400, D), jnp.float32),
        jax.ShapeDtypeStruct((512, D), jnp.float32),
        jax.ShapeDtypeStruct((4096, D), jnp.float32),
        jax.ShapeDtypeStruct((N_COLS,), jnp.int32),
        jax.ShapeDtypeStruct((N_COLS,), jnp.int32),
    ],
)


def _sc_body(lti_hbm, fei_hbm, pi_hbm, plt_hbm, pfe_hbm, ppair_hbm, out_hbm,
             ia1, ia2, ia3, g1a, g2a, g3a, g1b, g2b, g3b,
             sh_lt, sh_fe,
             s1a, s2a, s3a, s1b, s2b, s3b, osa, osb):
    cid = lax.axis_index("c")
    sid = lax.axis_index("s")
    wid = sid * NC + cid
    base = wid * ROWS_PER_W

    @pl.when(sid == 0)
    def _():
        pltpu.sync_copy(plt_hbm, sh_lt)
        pltpu.sync_copy(pfe_hbm, sh_fe)

    pltpu.sync_copy(lti_hbm.at[pl.ds(base, ROWS_PER_W)], ia1)
    pltpu.sync_copy(fei_hbm.at[pl.ds(base, ROWS_PER_W)], ia2)
    pltpu.sync_copy(pi_hbm.at[pl.ds(base, ROWS_PER_W)], ia3)
    plsc.subcore_barrier()

    def g_copies(c, bufs, sems):
        off = pl.multiple_of(c * CH, CH)
        srcs = (sh_lt.at[ia1.at[pl.ds(off, CH)]],
                sh_fe.at[ia2.at[pl.ds(off, CH)]],
                ppair_hbm.at[ia3.at[pl.ds(off, CH)]])
        return [pltpu.make_async_copy(s, b, m)
                for s, b, m in zip(srcs, bufs, sems)]

    def issue(c, bufs, sems):
        for cp in g_copies(c, bufs, sems):
            cp.start()

    def wait_g(c, bufs, sems):
        for cp in g_copies(c, bufs, sems):
            cp.wait()

    def out_copy(c, g1, osem):
        off = pl.multiple_of(c * CH, CH)
        return pltpu.make_async_copy(
            g1, out_hbm.at[pl.ds(base + off, CH)], osem)

    def compute(g1, g2, g3):
        def row(r, carry):
            for j in range(D // 16):
                sl = pl.ds(j * 16, 16)
                plsc.addupdate(g1.at[r, sl], g2[r, sl] + g3[r, sl])
            return carry
        lax.fori_loop(0, CH, row, 0)

    bufs_a, sems_a = (g1a, g2a, g3a), (s1a, s2a, s3a)
    bufs_b, sems_b = (g1b, g2b, g3b), (s1b, s2b, s3b)

    issue(0, bufs_a, sems_a)

    def pair(i, carry):
        c0 = 2 * i
        wait_g(c0, bufs_a, sems_a)

        @pl.when(i > 0)
        def _():
            out_copy(c0 - 1, g1b, osb).wait()

        issue(c0 + 1, bufs_b, sems_b)
        compute(g1a, g2a, g3a)
        out_copy(c0, g1a, osa).start()
        wait_g(c0 + 1, bufs_b, sems_b)
        out_copy(c0, g1a, osa).wait()

        @pl.when(i < NPAIR - 1)
        def _():
            issue(c0 + 2, bufs_a, sems_a)

        compute(g1b, g2b, g3b)
        out_copy(c0 + 1, g1b, osb).start()
        return carry

    lax.fori_loop(0, NPAIR, pair, 0)
    out_copy(NCHUNK - 1, g1b, osb).wait()


_sc_call = functools.partial(
    pl.kernel,
    mesh=plsc.VectorSubcoreMesh(core_axis_name="c", subcore_axis_name="s"),
    out_type=jax.ShapeDtypeStruct((N_COLS, D), jnp.float32),
    scratch_types=[
        pltpu.VMEM((ROWS_PER_W,), jnp.int32),
        pltpu.VMEM((ROWS_PER_W,), jnp.int32),
        pltpu.VMEM((ROWS_PER_W,), jnp.int32),
        pltpu.VMEM((CH, D), jnp.float32),
        pltpu.VMEM((CH, D), jnp.float32),
        pltpu.VMEM((CH, D), jnp.float32),
        pltpu.VMEM((CH, D), jnp.float32),
        pltpu.VMEM((CH, D), jnp.float32),
        pltpu.VMEM((CH, D), jnp.float32),
        pltpu.VMEM_SHARED((400, D), jnp.float32),
        pltpu.VMEM_SHARED((512, D), jnp.float32),
        pltpu.SemaphoreType.DMA,
        pltpu.SemaphoreType.DMA,
        pltpu.SemaphoreType.DMA,
        pltpu.SemaphoreType.DMA,
        pltpu.SemaphoreType.DMA,
        pltpu.SemaphoreType.DMA,
        pltpu.SemaphoreType.DMA,
        pltpu.SemaphoreType.DMA,
    ],
)(_sc_body)


def kernel(level_indices, type_indices, feature_indices, exchange_indices,
           pair_indices, level_table, type_table, feature_table,
           exchange_table, pair_table, W, b):
    lt, fe, pp, lti, fei = _proj_call(
        level_table, type_table, feature_table, exchange_table, pair_table,
        W, b.reshape(1, D),
        level_indices.astype(jnp.int32), type_indices.astype(jnp.int32),
        feature_indices.astype(jnp.int32), exchange_indices.astype(jnp.int32))
    out = _sc_call(lti, fei, pair_indices.astype(jnp.int32), lt, fe, pp)
    return out
